# baseline (speedup 1.0000x reference)
import functools

import jax
import jax.numpy as jnp
from jax import lax
from jax.experimental import pallas as pl
from jax.experimental.pallas import tpu as pltpu
from jax.experimental.pallas import tpu_sc as plsc

_NC, _NS, _LANES = 2, 16, 16


def _build_sc_histogram(B, L, vocab, chunk, half, n_halves):
    rows_here = B // n_halves
    rows_per_w = rows_here // _NS
    n_chunks = rows_per_w // chunk
    groups = chunk // _LANES
    mesh = plsc.VectorSubcoreMesh(
        core_axis_name="c", subcore_axis_name="s",
        num_cores=1, num_subcores=_NS)

    @functools.partial(
        pl.kernel,
        out_type=jax.ShapeDtypeStruct((rows_here, vocab), jnp.float32),
        mesh=mesh,
        compiler_params=pltpu.CompilerParams(
            needs_layout_passes=False, use_tc_tiling_on_sc=False),
        scratch_types=[
            pltpu.VMEM((chunk * L,), jnp.int32),
            pltpu.VMEM((chunk,), jnp.int32),
            pltpu.VMEM((chunk, vocab), jnp.float32),
        ],
    )
    def sc_histogram(x_hbm, len_hbm, counts_hbm, x_v, len_v, counts_v):
        wid = lax.axis_index("s")
        lane = lax.iota(jnp.int32, _LANES)
        ones = jnp.ones((_LANES,), jnp.float32)
        zeros = jnp.zeros((_LANES,), jnp.float32)
        base0 = wid * rows_per_w
        for ci in range(n_chunks):
            base = base0 + ci * chunk
            gbase = half * rows_here + base
            pltpu.sync_copy(x_hbm.at[pl.ds(gbase * L, chunk * L)], x_v)
            pltpu.sync_copy(len_hbm.at[pl.ds(gbase, chunk)], len_v)

            @pl.loop(0, chunk, unroll=8)
            def _(r):
                for cc in range(vocab // _LANES):
                    counts_v[r, pl.ds(cc * _LANES, _LANES)] = zeros

            rows = [g * _LANES + lane for g in range(groups)]
            fbase = [r * L for r in rows]
            lens = [len_v[pl.ds(g * _LANES, _LANES)] for g in range(groups)]

            @pl.loop(0, L, unroll=2)
            def _(j):
                toks = [plsc.load_gather(x_v, [fbase[g] + j])
                        for g in range(groups)]
                masks = [lens[g] > j for g in range(groups)]
                for g in range(groups):
                    plsc.addupdate_scatter(
                        counts_v, [rows[g], toks[g]], ones, mask=masks[g])

            pltpu.sync_copy(counts_v, counts_hbm.at[pl.ds(base, chunk)])

    return sc_histogram


def _mlp_body(ca_ref, cb_ref, len_ref, emb_ref, w1_ref, b1_ref, w2_ref,
              b2_ref, out_ref):
    i = pl.program_id(0)
    nb = pl.num_programs(0)
    counts = jnp.where(i < nb // 2, ca_ref[...], cb_ref[...])
    hi = jax.lax.Precision.HIGHEST
    s = jnp.dot(counts, emb_ref[...],
                preferred_element_type=jnp.float32, precision=hi)
    rep = s / (len_ref[...] + 1e-8)
    h = jnp.dot(rep, w1_ref[...],
                preferred_element_type=jnp.float32, precision=hi)
    h = jnp.maximum(h + b1_ref[...], 0.0)
    out = jnp.dot(h, w2_ref[...],
                  preferred_element_type=jnp.float32, precision=hi)
    out_ref[...] = out + b2_ref[...]


def kernel(x, lengths, emb, w1, b1, w2, b2):
    B, L = x.shape
    vocab, dim = emb.shape
    hid, out_d = w2.shape[0], w2.shape[1]

    x_flat = jnp.reshape(x.astype(jnp.int32), (B * L,))
    len_i = lengths.astype(jnp.int32)
    counts_a = _build_sc_histogram(B, L, vocab, 128, 0, 2)(x_flat, len_i)
    counts_b = _build_sc_histogram(B, L, vocab, 128, 1, 2)(x_flat, len_i)

    lenf = lengths.astype(jnp.float32).reshape(B, 1)
    bt = 2048
    nb = B // bt
    hb = nb // 2
    logits = pl.pallas_call(
        _mlp_body,
        grid=(nb,),
        in_specs=[
            pl.BlockSpec((bt, vocab), lambda i: (jnp.minimum(i, hb - 1), 0)),
            pl.BlockSpec((bt, vocab),
                         lambda i: (jnp.maximum(i - hb, 0), 0)),
            pl.BlockSpec((bt, 1), lambda i: (i, 0)),
            pl.BlockSpec((vocab, dim), lambda i: (0, 0)),
            pl.BlockSpec((dim, hid), lambda i: (0, 0)),
            pl.BlockSpec((1, hid), lambda i: (0, 0)),
            pl.BlockSpec((hid, out_d), lambda i: (0, 0)),
            pl.BlockSpec((1, out_d), lambda i: (0, 0)),
        ],
        out_specs=pl.BlockSpec((bt, out_d), lambda i: (i, 0)),
        out_shape=jax.ShapeDtypeStruct((B, out_d), jnp.float32),
    )(counts_a, counts_b, lenf, emb, w1, b1.reshape(1, hid), w2,
      b2.reshape(1, out_d))
    return logits


# speedup vs baseline: 1.1602x; 1.1602x over previous
import functools

import jax
import jax.numpy as jnp
from jax import lax
from jax.experimental import pallas as pl
from jax.experimental.pallas import tpu as pltpu
from jax.experimental.pallas import tpu_sc as plsc

_NC, _NS, _LANES = 2, 16, 16
_NW = _NC * _NS


def _build_sc_histogram(B, L, vocab, chunk):
    rows_per_w = B // _NW
    n_chunks = rows_per_w // chunk
    groups = chunk // _LANES
    mesh = plsc.VectorSubcoreMesh(
        core_axis_name="c", subcore_axis_name="s",
        num_cores=_NC, num_subcores=_NS)

    @functools.partial(
        pl.kernel,
        out_type=jax.ShapeDtypeStruct((B, vocab), jnp.float32),
        mesh=mesh,
        compiler_params=pltpu.CompilerParams(
            needs_layout_passes=False, use_tc_tiling_on_sc=False),
        scratch_types=[
            pltpu.VMEM((chunk, L + 1), jnp.int32),
            pltpu.VMEM((chunk,), jnp.int32),
            pltpu.VMEM((chunk, vocab), jnp.float32),
        ],
    )
    def sc_histogram(x_hbm, len_hbm, counts_hbm, x_v, len_v, counts_v):
        wid = lax.axis_index("s") * _NC + lax.axis_index("c")
        lane = lax.iota(jnp.int32, _LANES)
        ones = jnp.ones((_LANES,), jnp.float32)
        zeros = jnp.zeros((_LANES,), jnp.float32)
        base0 = wid * rows_per_w
        for ci in range(n_chunks):
            base = base0 + ci * chunk
            pltpu.sync_copy(x_hbm.at[pl.ds(base, chunk), :],
                            x_v.at[:, pl.ds(0, L)])
            pltpu.sync_copy(len_hbm.at[pl.ds(base, chunk)], len_v)

            @pl.loop(0, chunk, unroll=8)
            def _(r):
                for cc in range(vocab // _LANES):
                    counts_v[r, pl.ds(cc * _LANES, _LANES)] = zeros

            rows = [g * _LANES + lane for g in range(groups)]
            lens = [len_v[pl.ds(g * _LANES, _LANES)] for g in range(groups)]

            @pl.loop(0, L, unroll=2)
            def _(j):
                toks = [plsc.load_gather(x_v, [rows[g], lane * 0 + j])
                        for g in range(groups)]
                masks = [lens[g] > j for g in range(groups)]
                for g in range(groups):
                    plsc.addupdate_scatter(
                        counts_v, [rows[g], toks[g]], ones, mask=masks[g])

            pltpu.sync_copy(counts_v, counts_hbm.at[pl.ds(base, chunk)])

    return sc_histogram


def _mlp_body(counts_ref, len_ref, emb_ref, w1_ref, b1_ref, w2_ref, b2_ref,
              out_ref):
    hi = jax.lax.Precision.HIGHEST
    s = jnp.dot(counts_ref[...], emb_ref[...],
                preferred_element_type=jnp.float32, precision=hi)
    rep = s / (len_ref[...] + 1e-8)
    h = jnp.dot(rep, w1_ref[...],
                preferred_element_type=jnp.float32, precision=hi)
    h = jnp.maximum(h + b1_ref[...], 0.0)
    out = jnp.dot(h, w2_ref[...],
                  preferred_element_type=jnp.float32, precision=hi)
    out_ref[...] = out + b2_ref[...]


def kernel(x, lengths, emb, w1, b1, w2, b2):
    B, L = x.shape
    vocab, dim = emb.shape
    hid, out_d = w2.shape[0], w2.shape[1]

    counts = _build_sc_histogram(B, L, vocab, chunk=128)(
        x.astype(jnp.int32), lengths.astype(jnp.int32))

    lenf = lengths.astype(jnp.float32).reshape(B, 1)
    bt = 2048
    logits = pl.pallas_call(
        _mlp_body,
        grid=(B // bt,),
        in_specs=[
            pl.BlockSpec((bt, vocab), lambda i: (i, 0)),
            pl.BlockSpec((bt, 1), lambda i: (i, 0)),
            pl.BlockSpec((vocab, dim), lambda i: (0, 0)),
            pl.BlockSpec((dim, hid), lambda i: (0, 0)),
            pl.BlockSpec((1, hid), lambda i: (0, 0)),
            pl.BlockSpec((hid, out_d), lambda i: (0, 0)),
            pl.BlockSpec((1, out_d), lambda i: (0, 0)),
        ],
        out_specs=pl.BlockSpec((bt, out_d), lambda i: (i, 0)),
        out_shape=jax.ShapeDtypeStruct((B, out_d), jnp.float32),
    )(counts, lenf, emb, w1, b1.reshape(1, hid), w2, b2.reshape(1, out_d))
    return logits


# speedup vs baseline: 1.2110x; 1.0438x over previous
import functools

import jax
import jax.numpy as jnp
from jax import lax
from jax.experimental import pallas as pl
from jax.experimental.pallas import tpu as pltpu
from jax.experimental.pallas import tpu_sc as plsc

_NC, _NS, _LANES = 2, 16, 16
_NW = _NC * _NS


def _build_sc_histogram(B, L, vocab, chunk):
    rows_per_w = B // _NW
    n_chunks = rows_per_w // chunk
    groups = chunk // _LANES
    mesh = plsc.VectorSubcoreMesh(
        core_axis_name="c", subcore_axis_name="s",
        num_cores=_NC, num_subcores=_NS)

    @functools.partial(
        pl.kernel,
        out_type=jax.ShapeDtypeStruct((B, vocab), jnp.float32),
        mesh=mesh,
        compiler_params=pltpu.CompilerParams(
            needs_layout_passes=False, use_tc_tiling_on_sc=False),
        scratch_types=[
            pltpu.VMEM((chunk * L,), jnp.int32),
            pltpu.VMEM((chunk,), jnp.int32),
            pltpu.VMEM((chunk, vocab), jnp.float32),
        ],
    )
    def sc_histogram(x_hbm, len_hbm, counts_hbm, x_v, len_v, counts_v):
        wid = lax.axis_index("s") * _NC + lax.axis_index("c")
        lane = lax.iota(jnp.int32, _LANES)
        ones = jnp.ones((_LANES,), jnp.float32)
        zeros = jnp.zeros((_LANES,), jnp.float32)
        base0 = wid * rows_per_w
        for ci in range(n_chunks):
            base = base0 + ci * chunk
            pltpu.sync_copy(x_hbm.at[pl.ds(base * L, chunk * L)], x_v)
            pltpu.sync_copy(len_hbm.at[pl.ds(base, chunk)], len_v)

            @pl.loop(0, chunk, unroll=8)
            def _(r):
                for cc in range(vocab // _LANES):
                    counts_v[r, pl.ds(cc * _LANES, _LANES)] = zeros

            rows = [g * _LANES + lane for g in range(groups)]
            fbase = [r * L for r in rows]
            lens = [len_v[pl.ds(g * _LANES, _LANES)] for g in range(groups)]

            @pl.loop(0, L, unroll=2)
            def _(j):
                toks = [plsc.load_gather(x_v, [fbase[g] + j])
                        for g in range(groups)]
                masks = [lens[g] > j for g in range(groups)]
                for g in range(groups):
                    plsc.addupdate_scatter(
                        counts_v, [rows[g], toks[g]], ones, mask=masks[g])

            pltpu.sync_copy(counts_v, counts_hbm.at[pl.ds(base, chunk)])

    return sc_histogram


def _mlp_body(counts_ref, len_ref, emb_ref, w1_ref, b1_ref, w2_ref, b2_ref,
              out_ref):
    hi = jax.lax.Precision.HIGHEST
    s = jnp.dot(counts_ref[...], emb_ref[...],
                preferred_element_type=jnp.float32, precision=hi)
    rep = s / (len_ref[...] + 1e-8)
    h = jnp.dot(rep, w1_ref[...],
                preferred_element_type=jnp.float32, precision=hi)
    h = jnp.maximum(h + b1_ref[...], 0.0)
    out = jnp.dot(h, w2_ref[...],
                  preferred_element_type=jnp.float32, precision=hi)
    out_ref[...] = out + b2_ref[...]


def kernel(x, lengths, emb, w1, b1, w2, b2):
    B, L = x.shape
    vocab, dim = emb.shape
    hid, out_d = w2.shape[0], w2.shape[1]

    x_flat = jnp.reshape(x.astype(jnp.int32), (B * L,)) & 0xFF
    counts = _build_sc_histogram(B, L, vocab, chunk=128)(
        x_flat, lengths.astype(jnp.int32))

    lenf = lengths.astype(jnp.float32).reshape(B, 1)
    bt = 2048
    logits = pl.pallas_call(
        _mlp_body,
        grid=(B // bt,),
        in_specs=[
            pl.BlockSpec((bt, vocab), lambda i: (i, 0)),
            pl.BlockSpec((bt, 1), lambda i: (i, 0)),
            pl.BlockSpec((vocab, dim), lambda i: (0, 0)),
            pl.BlockSpec((dim, hid), lambda i: (0, 0)),
            pl.BlockSpec((1, hid), lambda i: (0, 0)),
            pl.BlockSpec((hid, out_d), lambda i: (0, 0)),
            pl.BlockSpec((1, out_d), lambda i: (0, 0)),
        ],
        out_specs=pl.BlockSpec((bt, out_d), lambda i: (i, 0)),
        out_shape=jax.ShapeDtypeStruct((B, out_d), jnp.float32),
    )(counts, lenf, emb, w1, b1.reshape(1, hid), w2, b2.reshape(1, out_d))
    return logits


# speedup vs baseline: 1.3537x; 1.1178x over previous
import functools

import jax
import jax.numpy as jnp
from jax import lax
from jax.experimental import pallas as pl
from jax.experimental.pallas import tpu as pltpu
from jax.experimental.pallas import tpu_sc as plsc

_NC, _NS, _LANES = 2, 16, 16
_NW = _NC * _NS


def _build_sc_histogram(B, L, vocab, chunk):
    rows_per_w = B // _NW
    n_chunks = rows_per_w // chunk
    groups = chunk // _LANES
    mesh = plsc.VectorSubcoreMesh(
        core_axis_name="c", subcore_axis_name="s",
        num_cores=_NC, num_subcores=_NS)

    @functools.partial(
        pl.kernel,
        out_type=jax.ShapeDtypeStruct((B, vocab), jnp.float32),
        mesh=mesh,
        compiler_params=pltpu.CompilerParams(
            needs_layout_passes=False, use_tc_tiling_on_sc=False),
        scratch_types=[
            pltpu.VMEM((2, chunk * L), jnp.int32),
            pltpu.VMEM((2, chunk), jnp.int32),
            pltpu.VMEM((2, chunk, vocab), jnp.float32),
            pltpu.SemaphoreType.DMA,
            pltpu.SemaphoreType.DMA,
            pltpu.SemaphoreType.DMA,
            pltpu.SemaphoreType.DMA,
            pltpu.SemaphoreType.DMA,
            pltpu.SemaphoreType.DMA,
        ],
    )
    def sc_histogram(x_hbm, len_hbm, counts_hbm, x_v, len_v, counts_v,
                     sx0, sx1, sl0, sl1, so0, so1):
        wid = lax.axis_index("s") * _NC + lax.axis_index("c")
        lane = lax.iota(jnp.int32, _LANES)
        ones = jnp.ones((_LANES,), jnp.float32)
        zeros = jnp.zeros((_LANES,), jnp.float32)
        base0 = wid * rows_per_w
        sxs, sls, sos = [sx0, sx1], [sl0, sl1], [so0, so1]

        def start_in(ci):
            b = ci % 2
            base = base0 + ci * chunk
            dx = pltpu.async_copy(
                x_hbm.at[pl.ds(base * L, chunk * L)], x_v.at[b], sxs[b])
            dl = pltpu.async_copy(
                len_hbm.at[pl.ds(base, chunk)], len_v.at[b], sls[b])
            return dx, dl

        pend_in = {0: start_in(0)}
        pend_out = {}
        for ci in range(n_chunks):
            b = ci % 2
            base = base0 + ci * chunk
            if ci + 1 < n_chunks:
                pend_in[ci + 1] = start_in(ci + 1)
            dx, dl = pend_in.pop(ci)
            dx.wait()
            dl.wait()
            if ci - 2 in pend_out:
                pend_out.pop(ci - 2).wait()

            @pl.loop(0, chunk, unroll=8)
            def _(r):
                for cc in range(vocab // _LANES):
                    counts_v[b, r, pl.ds(cc * _LANES, _LANES)] = zeros

            rows = [g * _LANES + lane for g in range(groups)]
            fbase = [r * L for r in rows]
            lens = [len_v[b, pl.ds(g * _LANES, _LANES)]
                    for g in range(groups)]

            @pl.loop(0, L, unroll=4)
            def _(j):
                toks = [plsc.load_gather(x_v.at[b], [fbase[g] + j])
                        for g in range(groups)]
                masks = [lens[g] > j for g in range(groups)]
                for g in range(groups):
                    plsc.addupdate_scatter(
                        counts_v.at[b], [rows[g], toks[g]], ones,
                        mask=masks[g])

            pend_out[ci] = pltpu.async_copy(
                counts_v.at[b], counts_hbm.at[pl.ds(base, chunk)], sos[b])
        for d in pend_out.values():
            d.wait()

    return sc_histogram


def _mlp_body(counts_ref, len_ref, emb_ref, w1_ref, b1_ref, w2_ref, b2_ref,
              out_ref):
    hi = jax.lax.Precision.HIGHEST
    s = jnp.dot(counts_ref[...], emb_ref[...],
                preferred_element_type=jnp.float32, precision=hi)
    rep = s / (len_ref[...] + 1e-8)
    h = jnp.dot(rep, w1_ref[...],
                preferred_element_type=jnp.float32, precision=hi)
    h = jnp.maximum(h + b1_ref[...], 0.0)
    out = jnp.dot(h, w2_ref[...],
                  preferred_element_type=jnp.float32, precision=hi)
    out_ref[...] = out + b2_ref[...]


def kernel(x, lengths, emb, w1, b1, w2, b2):
    B, L = x.shape
    vocab, dim = emb.shape
    hid, out_d = w2.shape[0], w2.shape[1]

    x_flat = jnp.reshape(x.astype(jnp.int32), (B * L,))
    counts = _build_sc_histogram(B, L, vocab, chunk=128)(
        x_flat, lengths.astype(jnp.int32))

    lenf = lengths.astype(jnp.float32).reshape(B, 1)
    bt = 2048
    logits = pl.pallas_call(
        _mlp_body,
        grid=(B // bt,),
        in_specs=[
            pl.BlockSpec((bt, vocab), lambda i: (i, 0)),
            pl.BlockSpec((bt, 1), lambda i: (i, 0)),
            pl.BlockSpec((vocab, dim), lambda i: (0, 0)),
            pl.BlockSpec((dim, hid), lambda i: (0, 0)),
            pl.BlockSpec((1, hid), lambda i: (0, 0)),
            pl.BlockSpec((hid, out_d), lambda i: (0, 0)),
            pl.BlockSpec((1, out_d), lambda i: (0, 0)),
        ],
        out_specs=pl.BlockSpec((bt, out_d), lambda i: (i, 0)),
        out_shape=jax.ShapeDtypeStruct((B, out_d), jnp.float32),
    )(counts, lenf, emb, w1, b1.reshape(1, hid), w2, b2.reshape(1, out_d))
    return logits
